# baseline (speedup 1.0000x reference)
import jax
import jax.numpy as jnp
from jax.experimental import pallas as pl
from jax.experimental.pallas import tpu as pltpu

_H = 32
_W = 32
_D = 256


def _body(row_ref, col_ref, out_ref, pos_ref):
    b = pl.program_id(0)

    @pl.when(b == 0)
    def _():
        ce = col_ref[:_W, :]
        re = row_ref[:_H, :]
        j = jax.lax.broadcasted_iota(jnp.int32, (_W, _H * _W), 1)
        i = jax.lax.broadcasted_iota(jnp.int32, (_W, _H * _W), 0)
        sel_w = (j % _W == i).astype(jnp.float32)
        sel_h = (j // _W == i).astype(jnp.float32)
        dn = (((0,), (0,)), ((), ()))
        first = jax.lax.dot_general(
            ce, sel_w, dn,
            precision=jax.lax.Precision.HIGHEST,
            preferred_element_type=jnp.float32,
        )
        second = jax.lax.dot_general(
            re, sel_h, dn,
            precision=jax.lax.Precision.HIGHEST,
            preferred_element_type=jnp.float32,
        )
        pos_ref[:_D, :] = first
        pos_ref[_D:, :] = second

    out_ref[0] = pos_ref[...]


def kernel(x, row_embed, col_embed):
    b = x.shape[0]
    out = pl.pallas_call(
        _body,
        grid=(b,),
        in_specs=[
            pl.BlockSpec((50, _D), lambda i: (0, 0)),
            pl.BlockSpec((50, _D), lambda i: (0, 0)),
        ],
        out_specs=pl.BlockSpec((1, 2 * _D, _H * _W), lambda i: (i, 0, 0)),
        out_shape=jax.ShapeDtypeStruct((b, 2 * _D, _H * _W), jnp.float32),
        scratch_shapes=[pltpu.VMEM((2 * _D, _H * _W), jnp.float32)],
    )(row_embed, col_embed)
    return out.reshape(b, 2 * _D, _H, _W)


# speedup vs baseline: 1.0247x; 1.0247x over previous
import jax
import jax.numpy as jnp
from jax.experimental import pallas as pl
from jax.experimental.pallas import tpu as pltpu

_H = 32
_W = 32
_D = 256
_B = 8


def _body(row_ref, col_ref, out_hbm, pos_ref, sem):
    ce = col_ref[:_W, :]
    re = row_ref[:_H, :]
    j = jax.lax.broadcasted_iota(jnp.int32, (_W, _H * _W), 1)
    i = jax.lax.broadcasted_iota(jnp.int32, (_W, _H * _W), 0)
    sel_w = (j % _W == i).astype(jnp.float32)
    sel_h = (j // _W == i).astype(jnp.float32)
    dn = (((0,), (0,)), ((), ()))
    pos_ref[:_D, :] = jax.lax.dot_general(
        ce, sel_w, dn,
        precision=jax.lax.Precision.HIGHEST,
        preferred_element_type=jnp.float32,
    )
    pos_ref[_D:, :] = jax.lax.dot_general(
        re, sel_h, dn,
        precision=jax.lax.Precision.HIGHEST,
        preferred_element_type=jnp.float32,
    )
    copies = [
        pltpu.make_async_copy(pos_ref, out_hbm.at[b], sem.at[b])
        for b in range(_B)
    ]
    for c in copies:
        c.start()
    for c in copies:
        c.wait()


def kernel(x, row_embed, col_embed):
    b = x.shape[0]
    out = pl.pallas_call(
        _body,
        in_specs=[
            pl.BlockSpec(memory_space=pltpu.MemorySpace.VMEM),
            pl.BlockSpec(memory_space=pltpu.MemorySpace.VMEM),
        ],
        out_specs=pl.BlockSpec(memory_space=pltpu.MemorySpace.HBM),
        out_shape=jax.ShapeDtypeStruct((b, 2 * _D, _H * _W), jnp.float32),
        scratch_shapes=[
            pltpu.VMEM((2 * _D, _H * _W), jnp.float32),
            pltpu.SemaphoreType.DMA((_B,)),
        ],
    )(row_embed, col_embed)
    return out.reshape(b, 2 * _D, _H, _W)
